# Initial kernel scaffold; baseline (speedup 1.0000x reference)
import jax
import jax.numpy as jnp
from jax.experimental import pallas as pl


def kernel(tile_detections, tile_offsets):
    raise NotImplementedError("write your pallas kernel here")


# speedup vs baseline: 8.2627x; 8.2627x over previous
import jax
import jax.numpy as jnp
from jax import lax
from jax.experimental import pallas as pl
from jax.experimental.pallas import tpu as pltpu

_T = 25
_D = 1000
_N = _T * _D
_RB = 196
_NP = _RB * 128
_PAD = _NP - _N
_K = 1000
_KP = 1024
_OUT = 300
_OUTP = 384
_THR = 0.65
_NB = _KP // 128


def _nms_kernel(data_ref, shift_ref, out_ref, a_scr, keep_scr):
    f32 = jnp.float32
    i32 = jnp.int32
    x1 = data_ref[0] + shift_ref[0]
    y1 = data_ref[1] + shift_ref[1]
    x2 = data_ref[2] + shift_ref[2]
    y2 = data_ref[3] + shift_ref[3]
    conf = data_ref[4]
    labl = data_ref[5]

    cbits = lax.bitcast_convert_type(conf, i32)

    def bs_body(_, lohi):
        lo, hi = lohi
        mid = lo + (hi - lo) // 2
        cnt = jnp.sum((cbits > mid).astype(i32))
        big = cnt >= _K
        lo2 = jnp.where(big, mid, lo)
        hi2 = jnp.where(big, hi, mid)
        go = (hi - lo) > 1
        return (jnp.where(go, lo2, lo), jnp.where(go, hi2, hi))

    _, tau = lax.fori_loop(0, 32, bs_body, (i32(-1), i32(1 << 30)))
    m = jnp.sum((cbits > tau).astype(i32))
    e = (_K - m).astype(f32)

    gt = (cbits > tau).astype(f32)
    tie = (cbits == tau).astype(f32)

    lane_lt = (lax.broadcasted_iota(i32, (128, 128), 0)
               < lax.broadcasted_iota(i32, (128, 128), 1)).astype(f32)
    blk_lt = (lax.broadcasted_iota(i32, (_RB, _RB), 1)
              < lax.broadcasted_iota(i32, (_RB, _RB), 0)).astype(f32)
    eye_rb = (lax.broadcasted_iota(i32, (_RB, _RB), 0)
              == lax.broadcasted_iota(i32, (_RB, _RB), 1)).astype(f32)
    eye_kp = (lax.broadcasted_iota(i32, (_KP, _KP), 0)
              == lax.broadcasted_iota(i32, (_KP, _KP), 1)).astype(f32)

    hp = lax.Precision.HIGHEST

    def tr_rb(v):
        return lax.dot_general(v, eye_rb, (((0,), (0,)), ((), ())),
                               preferred_element_type=f32, precision=hp)

    def tr_kp(v):
        return lax.dot_general(v, eye_kp, (((0,), (0,)), ((), ())),
                               preferred_element_type=f32, precision=hp)

    def mm(a, b):
        return jnp.dot(a, b, preferred_element_type=f32)

    def mmx(a, b):
        return jnp.dot(a, b, preferred_element_type=f32, precision=hp)

    tie_rank = mm(tie, lane_lt) + mm(blk_lt, jnp.sum(tie, axis=1, keepdims=True))
    sel = gt + tie * (tie_rank < e).astype(f32)
    within = mm(sel, lane_lt)
    rowcnt = jnp.sum(sel, axis=1, keepdims=True)
    before = mm(blk_lt, rowcnt)

    p_i = lax.broadcasted_iota(i32, (_KP, 1), 0).astype(f32)
    before_r = tr_rb(before)
    rowcnt_r = tr_rb(rowcnt)
    R = ((before_r <= p_i) & (p_i < before_r + rowcnt_r)).astype(f32)
    w = p_i - jnp.sum(R * before_r, axis=1, keepdims=True)
    RW = mm(R, within)
    RS = mm(R, sel)
    L = ((RW == w) & (RS > 0.5)).astype(f32)

    def gather(field):
        return jnp.sum(L * mmx(R, field), axis=1, keepdims=True)

    cx1, cy1, cx2, cy2 = gather(x1), gather(y1), gather(x2), gather(y2)
    cconf, clab = gather(conf), gather(labl)
    lane_f = lax.broadcasted_iota(i32, (1, 128), 1).astype(f32)
    blk_f = lax.broadcasted_iota(i32, (_RB, 128), 0).astype(f32)
    cidx = (jnp.sum(R * lax.broadcasted_iota(i32, (1, _RB), 1).astype(f32),
                    axis=1, keepdims=True) * 128.0
            + jnp.sum(L * lane_f, axis=1, keepdims=True))

    valid_t = lax.broadcasted_iota(i32, (_KP, 1), 0) < _K
    valid_s = lax.broadcasted_iota(i32, (1, _KP), 1) < _K

    cs, is_ = tr_kp(cconf), tr_kp(cidx)
    Gf = (((cconf > cs) | ((cconf == cs) & (cidx < is_))) & valid_t).astype(f32)
    rank = jnp.sum(Gf, axis=0, keepdims=True)

    slot_i = lax.broadcasted_iota(i32, (_KP, 1), 0).astype(f32)
    P = (rank == slot_i).astype(f32) * valid_s.astype(f32)
    sx1, sy1, sx2, sy2 = mmx(P, cx1), mmx(P, cy1), mmx(P, cx2), mmx(P, cy2)
    sconf, slab = mmx(P, cconf), mmx(P, clab)

    mc = jnp.maximum(jnp.maximum(jnp.max(cx1), jnp.max(cy1)),
                     jnp.maximum(jnp.max(cx2), jnp.max(cy2))) + 1.0
    ox1, oy1 = sx1 + slab * mc, sy1 + slab * mc
    ox2, oy2 = sx2 + slab * mc, sy2 + slab * mc
    tx1, ty1, tx2, ty2 = tr_kp(ox1), tr_kp(oy1), tr_kp(ox2), tr_kp(oy2)
    area_c = (ox2 - ox1) * (oy2 - oy1)
    area_r = tr_kp(area_c)
    xx1 = jnp.maximum(ox1, tx1)
    yy1 = jnp.maximum(oy1, ty1)
    xx2 = jnp.minimum(ox2, tx2)
    yy2 = jnp.minimum(oy2, ty2)
    inter = jnp.maximum(xx2 - xx1, 0.0) * jnp.maximum(yy2 - yy1, 0.0)
    union = area_c + area_r - inter
    a_scr[...] = (inter / (union + 1e-9) > _THR).astype(f32)
    keep_scr[...] = valid_s.astype(f32)

    col_g = lax.broadcasted_iota(i32, (1, _KP), 1).astype(f32)

    for b in range(_NB):
        base = b * 128
        kb0 = keep_scr[:, base:base + 128]

        def inner(j, kb):
            a8 = a_scr[pl.ds(base + j * 8, 8), base:base + 128]
            for r in range(8):
                arow = a8[r:r + 1, :]
                fi = (j * 8 + r).astype(f32)
                ki = jnp.sum(kb * (lane_f == fi).astype(f32))
                sup = arow * (lane_f > fi).astype(f32) * ki
                kb = kb * (1.0 - sup)
            return kb

        kb = lax.fori_loop(0, 16, inner, kb0)
        keep_scr[:, base:base + 128] = kb
        if b + 1 < _NB:
            hits = mm(kb, a_scr[base:base + 128, :])
            supv = ((hits > 0.0) & (col_g > (base + 127.0))).astype(f32)
            keep_scr[...] = keep_scr[...] * (1.0 - supv)

    keep = keep_scr[...]
    t_lt_s = (lax.broadcasted_iota(i32, (_KP, _KP), 0)
              < lax.broadcasted_iota(i32, (_KP, _KP), 1)).astype(f32)
    pos2 = mm(keep, t_lt_s)
    r_i = lax.broadcasted_iota(i32, (_OUTP, 1), 0).astype(f32)
    oh2 = (pos2 == r_i).astype(f32) * keep

    for c, scol in enumerate((sx1, sy1, sx2, sy2, sconf, slab)):
        out_ref[c:c + 1, :] = lax.dot_general(
            scol, oh2, (((0,), (1,)), ((), ())),
            preferred_element_type=f32, precision=hp)
    out_ref[6:8, :] = jnp.zeros((2, _OUTP), f32)


def kernel(tile_detections, tile_offsets):
    f32 = jnp.float32
    dets = tile_detections.astype(f32).reshape(_N, 6)
    pad = jnp.zeros((_PAD, 6), f32).at[:, 4].set(-1.0)
    data = jnp.concatenate([dets, pad], axis=0).T.reshape(6, _RB, 128)

    off = tile_offsets.astype(f32)
    shift6 = jnp.concatenate([off, off, jnp.zeros((_T, 2), f32)], axis=1)
    shiftP = jnp.concatenate(
        [jnp.broadcast_to(shift6[:, None, :], (_T, _D, 6)).reshape(_N, 6),
         jnp.zeros((_PAD, 6), f32)], axis=0)
    shift = shiftP.T.reshape(6, _RB, 128)

    out = pl.pallas_call(
        _nms_kernel,
        out_shape=jax.ShapeDtypeStruct((8, _OUTP), f32),
        scratch_shapes=[
            pltpu.VMEM((_KP, _KP), f32),
            pltpu.VMEM((1, _KP), f32),
        ],
    )(data, shift)
    return out[:6, :_OUT].T


# speedup vs baseline: 9.3409x; 1.1305x over previous
import jax
import jax.numpy as jnp
from jax import lax
from jax.experimental import pallas as pl
from jax.experimental.pallas import tpu as pltpu

_T = 25
_D = 1000
_N = _T * _D
_RB = 196
_NP = _RB * 128
_PAD = _NP - _N
_K = 1000
_KP = 1024
_OUT = 300
_OUTP = 384
_THR = 0.65
_NB = _KP // 128


def _nms_kernel(data_ref, shift_ref, out_ref, a_scr, keep_scr):
    f32 = jnp.float32
    i32 = jnp.int32
    x1 = data_ref[0] + shift_ref[0]
    y1 = data_ref[1] + shift_ref[1]
    x2 = data_ref[2] + shift_ref[2]
    y2 = data_ref[3] + shift_ref[3]
    conf = data_ref[4]
    labl = data_ref[5]

    cbits = lax.bitcast_convert_type(conf, i32)

    def bs_body(_, lohi):
        lo, hi = lohi
        mid = lo + (hi - lo) // 2
        cnt = jnp.sum((cbits > mid).astype(i32))
        big = cnt >= _K
        lo2 = jnp.where(big, mid, lo)
        hi2 = jnp.where(big, hi, mid)
        go = (hi - lo) > 1
        return (jnp.where(go, lo2, lo), jnp.where(go, hi2, hi))

    _, tau = lax.fori_loop(0, 32, bs_body, (i32(-1), i32(1 << 30)))
    m = jnp.sum((cbits > tau).astype(i32))
    e = (_K - m).astype(f32)

    gt = (cbits > tau).astype(f32)
    tie = (cbits == tau).astype(f32)

    lane_lt = (lax.broadcasted_iota(i32, (128, 128), 0)
               < lax.broadcasted_iota(i32, (128, 128), 1)).astype(f32)
    blk_lt = (lax.broadcasted_iota(i32, (_RB, _RB), 1)
              < lax.broadcasted_iota(i32, (_RB, _RB), 0)).astype(f32)
    eye_rb = (lax.broadcasted_iota(i32, (_RB, _RB), 0)
              == lax.broadcasted_iota(i32, (_RB, _RB), 1)).astype(f32)
    eye_kp = (lax.broadcasted_iota(i32, (_KP, _KP), 0)
              == lax.broadcasted_iota(i32, (_KP, _KP), 1)).astype(f32)

    hp = lax.Precision.HIGHEST

    def tr_rb(v):
        return lax.dot_general(v, eye_rb, (((0,), (0,)), ((), ())),
                               preferred_element_type=f32, precision=hp)

    def tr_kp(v):
        return lax.dot_general(v, eye_kp, (((0,), (0,)), ((), ())),
                               preferred_element_type=f32, precision=hp)

    def mm(a, b):
        return jnp.dot(a, b, preferred_element_type=f32)


    tie_rank = mm(tie, lane_lt) + mm(blk_lt, jnp.sum(tie, axis=1, keepdims=True))
    sel = gt + tie * (tie_rank < e).astype(f32)
    within = mm(sel, lane_lt)
    rowcnt = jnp.sum(sel, axis=1, keepdims=True)
    before = mm(blk_lt, rowcnt)

    p_i = lax.broadcasted_iota(i32, (_KP, 1), 0).astype(f32)
    before_r = tr_rb(before)
    rowcnt_r = tr_rb(rowcnt)
    R = ((before_r <= p_i) & (p_i < before_r + rowcnt_r)).astype(f32)
    w = p_i - jnp.sum(R * before_r, axis=1, keepdims=True)
    RW = mm(R, within)
    RS = mm(R, sel)
    L = ((RW == w) & (RS > 0.5)).astype(f32)

    X6 = jnp.concatenate([x1, y1, x2, y2, conf, labl], axis=1)
    bf16 = jnp.bfloat16
    Xa = X6.astype(bf16).astype(f32)
    rem = X6 - Xa
    Xb = rem.astype(bf16).astype(f32)
    Xc = rem - Xb

    def gat(Rm):
        return mm(Rm, Xa) + mm(Rm, Xb) + mm(Rm, Xc)

    def pick(G, c, Lm):
        return jnp.sum(Lm * G[:, c * 128:(c + 1) * 128], axis=1, keepdims=True)

    cG = gat(R)
    cconf = pick(cG, 4, L)
    lane_f = lax.broadcasted_iota(i32, (1, 128), 1).astype(f32)
    blk_r = lax.broadcasted_iota(i32, (1, _RB), 1).astype(f32)
    cb = jnp.sum(R * blk_r, axis=1, keepdims=True)
    cl = jnp.sum(L * lane_f, axis=1, keepdims=True)
    cidx = cb * 128.0 + cl

    valid_t = lax.broadcasted_iota(i32, (_KP, 1), 0) < _K
    valid_s = lax.broadcasted_iota(i32, (1, _KP), 1) < _K

    cs, is_ = tr_kp(cconf), tr_kp(cidx)
    Gf = (((cconf > cs) | ((cconf == cs) & (cidx < is_))) & valid_t).astype(f32)
    rank = jnp.sum(Gf, axis=0, keepdims=True)

    slot_i = lax.broadcasted_iota(i32, (_KP, 1), 0).astype(f32)
    P = (rank == slot_i).astype(f32) * valid_s.astype(f32)
    brank = mm(P, cb)
    lrank = mm(P, cl)
    R2 = (blk_r == brank).astype(f32) * valid_t.astype(f32)
    L2 = (lane_f == lrank).astype(f32)
    sG = gat(R2)
    sx1, sy1 = pick(sG, 0, L2), pick(sG, 1, L2)
    sx2, sy2 = pick(sG, 2, L2), pick(sG, 3, L2)
    sconf, slab = pick(sG, 4, L2), pick(sG, 5, L2)

    neg = jnp.float32(-3e38)
    selm = sel > 0.5
    mc = jnp.maximum(
        jnp.maximum(jnp.max(jnp.where(selm, x1, neg)),
                    jnp.max(jnp.where(selm, y1, neg))),
        jnp.maximum(jnp.max(jnp.where(selm, x2, neg)),
                    jnp.max(jnp.where(selm, y2, neg)))) + 1.0
    ox1, oy1 = sx1 + slab * mc, sy1 + slab * mc
    ox2, oy2 = sx2 + slab * mc, sy2 + slab * mc
    tx1, ty1, tx2, ty2 = tr_kp(ox1), tr_kp(oy1), tr_kp(ox2), tr_kp(oy2)
    area_c = (ox2 - ox1) * (oy2 - oy1)
    area_r = tr_kp(area_c)
    xx1 = jnp.maximum(ox1, tx1)
    yy1 = jnp.maximum(oy1, ty1)
    xx2 = jnp.minimum(ox2, tx2)
    yy2 = jnp.minimum(oy2, ty2)
    inter = jnp.maximum(xx2 - xx1, 0.0) * jnp.maximum(yy2 - yy1, 0.0)
    union = area_c + area_r - inter
    a_scr[...] = (inter / (union + 1e-9) > _THR).astype(f32)
    keep_scr[...] = valid_s.astype(f32)

    col_g = lax.broadcasted_iota(i32, (1, _KP), 1).astype(f32)

    for b in range(_NB):
        base = b * 128
        kb0 = keep_scr[:, base:base + 128]

        def inner(j, kb):
            a8 = a_scr[pl.ds(base + j * 8, 8), base:base + 128]
            for r in range(8):
                arow = a8[r:r + 1, :]
                fi = (j * 8 + r).astype(f32)
                ki = jnp.sum(kb * (lane_f == fi).astype(f32))
                sup = arow * (lane_f > fi).astype(f32) * ki
                kb = kb * (1.0 - sup)
            return kb

        kb = lax.fori_loop(0, 16, inner, kb0)
        keep_scr[:, base:base + 128] = kb
        if b + 1 < _NB:
            hits = mm(kb, a_scr[base:base + 128, :])
            supv = ((hits > 0.0) & (col_g > (base + 127.0))).astype(f32)
            keep_scr[...] = keep_scr[...] * (1.0 - supv)

    keep = keep_scr[...]
    t_lt_s = (lax.broadcasted_iota(i32, (_KP, _KP), 0)
              < lax.broadcasted_iota(i32, (_KP, _KP), 1)).astype(f32)
    pos2 = mm(keep, t_lt_s)
    r_i = lax.broadcasted_iota(i32, (_OUTP, 1), 0).astype(f32)
    oh2 = (pos2 == r_i).astype(f32) * keep

    for c, scol in enumerate((sx1, sy1, sx2, sy2, sconf, slab)):
        out_ref[c:c + 1, :] = lax.dot_general(
            scol, oh2, (((0,), (1,)), ((), ())),
            preferred_element_type=f32, precision=hp)
    out_ref[6:8, :] = jnp.zeros((2, _OUTP), f32)


def kernel(tile_detections, tile_offsets):
    f32 = jnp.float32
    dets = tile_detections.astype(f32).reshape(_N, 6)
    pad = jnp.zeros((_PAD, 6), f32).at[:, 4].set(-1.0)
    data = jnp.concatenate([dets, pad], axis=0).T.reshape(6, _RB, 128)

    off = tile_offsets.astype(f32)
    shift6 = jnp.concatenate([off, off, jnp.zeros((_T, 2), f32)], axis=1)
    shiftP = jnp.concatenate(
        [jnp.broadcast_to(shift6[:, None, :], (_T, _D, 6)).reshape(_N, 6),
         jnp.zeros((_PAD, 6), f32)], axis=0)
    shift = shiftP.T.reshape(6, _RB, 128)

    out = pl.pallas_call(
        _nms_kernel,
        out_shape=jax.ShapeDtypeStruct((8, _OUTP), f32),
        scratch_shapes=[
            pltpu.VMEM((_KP, _KP), f32),
            pltpu.VMEM((1, _KP), f32),
        ],
    )(data, shift)
    return out[:6, :_OUT].T
